# baseline (speedup 1.0000x reference)
import jax
import jax.numpy as jnp
import numpy as np
from jax.experimental import pallas as pl
from jax.experimental.pallas import tpu as pltpu

_B = 8
_C = 192
_H = 56
_NB = 4
_NH = 4
_HD = _C // _NH
_G = 4
_IPG = _C // _G
_HS = _H // 2
_LS = _HS * _HS
_L = _H * _H


def _fused_kernel(x_ref, cw_ref, qv_ref, cb_ref, bw_ref, bb_ref, o_ref,
                  wall_ref):
    f32 = jnp.float32

    @pl.when(pl.program_id(0) == 0)
    def _build_wall():
        def z(r, c):
            return jnp.zeros((r, c), f32)

        qp, kp, vp = [], [], []
        for i in range(_NB):
            c0, c1, c2, c3 = (cw_ref[i, g] for g in range(_G))
            qp.append(jnp.concatenate([
                jnp.concatenate([c0, z(_IPG, 144)], axis=1),
                jnp.concatenate([z(16, 48), c1[0:16], z(16, 96)], axis=1),
            ], axis=0))
            kp.append(jnp.concatenate([
                jnp.concatenate([z(32, 48), c1[16:48], z(32, 96)], axis=1),
                jnp.concatenate([z(32, 96), c2[0:32], z(32, 48)], axis=1),
            ], axis=0))
            vp.append(jnp.concatenate([
                jnp.concatenate([z(16, 96), c2[32:48], z(16, 48)], axis=1),
                jnp.concatenate([z(_IPG, 144), c3], axis=1),
            ], axis=0))
        wall_ref[...] = jnp.concatenate(qp + kp + vp, axis=0)

    xv = x_ref[0]
    wall = wall_ref[...]

    mt = jax.lax.dot_general(xv, wall, (((0,), (1,)), ((), ())),
                             preferred_element_type=f32)
    mt3 = mt.reshape(_HS, 2 * _H, _NB * _C)
    mv = jnp.maximum(mt3[:, :_H, :], mt3[:, _H:, :]).reshape(_HS * _H, _NB * _C)
    nv = jnp.minimum(mt3[:, :_H, :], mt3[:, _H:, :]).reshape(_HS * _H, _NB * _C)
    p2 = jnp.maximum(mv, jnp.roll(mv, -1, axis=0))
    n2 = jnp.minimum(nv, jnp.roll(nv, -1, axis=0))

    fs = []
    for i in range(_NB):
        acc = None
        for g in range(_G):
            cwg = cw_ref[i, g]
            wwt = jax.lax.dot_general(cwg, cwg, (((1,), (1,)), ((), ())),
                                      preferred_element_type=f32)
            rid = jax.lax.broadcasted_iota(jnp.int32, (_IPG, _IPG), 0)
            cid = jax.lax.broadcasted_iota(jnp.int32, (_IPG, _IPG), 1)
            dif = wwt - jnp.where(rid == cid, f32(1.0), f32(0.0))
            s = jnp.sum(dif * dif)
            acc = s if acc is None else acc + s
        fs.append(f32(1.0) - acc / f32(_G * _IPG * _IPG))

    mrow = jax.lax.broadcasted_iota(jnp.int32, (64, _NH), 0)
    hcol = jax.lax.broadcasted_iota(jnp.int32, (64, _NH), 1)
    smat_h = jnp.where(mrow // 16 == hcol, f32(1.0), f32(0.0))
    crow = jax.lax.broadcasted_iota(jnp.int32, (_C, 64), 0)
    mcol = jax.lax.broadcasted_iota(jnp.int32, (_C, 64), 1)
    rmat = jnp.where(crow // 3 == mcol, f32(1.0), f32(0.0))
    srow = jax.lax.broadcasted_iota(jnp.int32, (2 * _LS, 1), 0)
    even = (srow % 2) == 0

    isq = f32(1.0 / np.sqrt(_HD))
    K = jnp.zeros((_C, 1), f32)
    F = f32(1.0)
    for i in range(_NB):
        qcol = qv_ref[i, 0:_C, :]
        kcol = qv_ref[i, _C:2 * _C, :]
        vcol = qv_ref[i, 2 * _C:3 * _C, :]
        a_row = jax.lax.dot_general(qcol * kcol, rmat, (((0,), (0,)), ((), ())),
                                    preferred_element_type=f32) * isq
        mixk = jnp.concatenate(
            [jax.lax.dot_general(K[_IPG * g:_IPG * (g + 1), :], cw_ref[i, g],
                                 (((0,), (1,)), ((), ())),
                                 preferred_element_type=f32)
             for g in range(_G)], axis=1) + cb_ref[i]
        fp = jnp.maximum(F, f32(0.0))
        fn = jnp.minimum(F, f32(0.0))
        x_u = fp * p2[:, 64 * i:64 * i + 64] \
            + fn * n2[:, 64 * i:64 * i + 64] + mixk[:, 0:64]
        x_w = fp * p2[:, 256 + 64 * i:256 + 64 * i + 64] \
            + fn * n2[:, 256 + 64 * i:256 + 64 * i + 64] + mixk[:, 64:128]
        x_v = fp * p2[:, 512 + 64 * i:512 + 64 * i + 64] \
            + fn * n2[:, 512 + 64 * i:512 + 64 * i + 64] + mixk[:, 128:192]
        prod = x_u * x_w * a_row
        scores = jnp.dot(prod, smat_h, preferred_element_type=f32)
        scores = jnp.where(even, scores, f32(-1e30))
        mx = jnp.max(scores, axis=0, keepdims=True)
        e = jnp.exp(scores - mx)
        attn = e / jnp.sum(e, axis=0, keepdims=True)
        ws = jax.lax.dot_general(x_v, attn, (((0,), (0,)), ((), ())),
                                 preferred_element_type=f32)
        wsum = jnp.sum(ws * smat_h, axis=1, keepdims=True)
        ovec = vcol * jnp.dot(rmat, wsum, preferred_element_type=f32)
        constv = jnp.dot(bw_ref[i], ovec, preferred_element_type=f32) \
            + bb_ref[i]
        K = constv + fs[i] * K
        F = F * fs[i]

    o_ref[0] = F * xv + K


def kernel(x, cluster_w, cluster_b, qkv_w, bais_w, bais_b):
    f32 = jnp.float32
    xf = x.astype(f32).reshape(_B, _C, _L)
    qv = qkv_w.astype(f32).reshape(_NB, 3 * _C, 1)
    cb2 = cluster_b.astype(f32).reshape(_NB, 1, _C)
    bb3 = bais_b.astype(f32).reshape(_NB, _C, 1)

    out = pl.pallas_call(
        _fused_kernel,
        grid=(_B,),
        in_specs=[
            pl.BlockSpec((1, _C, _L), lambda b: (b, 0, 0)),
            pl.BlockSpec((_NB, _G, _IPG, _IPG), lambda b: (0, 0, 0, 0)),
            pl.BlockSpec((_NB, 3 * _C, 1), lambda b: (0, 0, 0)),
            pl.BlockSpec((_NB, 1, _C), lambda b: (0, 0, 0)),
            pl.BlockSpec((_NB, _C, _C), lambda b: (0, 0, 0)),
            pl.BlockSpec((_NB, _C, 1), lambda b: (0, 0, 0)),
        ],
        out_specs=pl.BlockSpec((1, _C, _L), lambda b: (b, 0, 0)),
        out_shape=jax.ShapeDtypeStruct((_B, _C, _L), f32),
        scratch_shapes=[pltpu.VMEM((_NB * _C, _C), f32)],
    )(xf, cluster_w.astype(f32), qv, cb2, bais_w.astype(f32), bb3)
    return out.reshape(_B, _C, _H, _H)
